# Initial kernel scaffold; baseline (speedup 1.0000x reference)
import jax
import jax.numpy as jnp
from jax.experimental import pallas as pl


def kernel(x, edge_index, edge_attr, W1, as1, ad1, We1, ae1, b1, W2, as2, ad2, We2, ae2, b2, L1, bL1, L2, bL2):
    raise NotImplementedError("write your pallas kernel here")


# speedup vs baseline: 7.5120x; 7.5120x over previous
import functools

import jax
import jax.numpy as jnp
from jax import lax
from jax.experimental import pallas as pl
from jax.experimental.pallas import tpu as pltpu
from jax.experimental.pallas import tpu_sc as plsc

_NC = 2
_NS = 16
_R = 896
_NPASS = 6
_NPAD = _NC * _NPASS * _R
_CH = 400
_ICH = 125
_EPS = 1e-16
_SC_PARAMS = pltpu.CompilerParams(needs_layout_passes=False,
                                  use_tc_tiling_on_sc=False)


def _mm_body(a_ref, b_ref, bias_ref, o_ref):
    o_ref[...] = jnp.dot(a_ref[...], b_ref[...],
                         preferred_element_type=jnp.float32) + bias_ref[...]


def _matmul(a, b, bias_row, bm):
    m, k = a.shape
    _, nn = b.shape
    grid = (m // bm,)
    return pl.pallas_call(
        _mm_body,
        grid=grid,
        in_specs=[pl.BlockSpec((bm, k), lambda i: (i, 0)),
                  pl.BlockSpec((k, nn), lambda i: (0, 0)),
                  pl.BlockSpec((1, nn), lambda i: (0, 0))],
        out_specs=pl.BlockSpec((bm, nn), lambda i: (i, 0)),
        out_shape=jax.ShapeDtypeStruct((m, nn), jnp.float32),
    )(a, b, bias_row)


def _head_body(a_ref, l1_ref, b1_ref, l2_ref, b2_ref, o_ref):
    t = jnp.dot(a_ref[...], l1_ref[...],
                preferred_element_type=jnp.float32) + b1_ref[...]
    t = jnp.maximum(t, 0.0)
    o_ref[...] = jnp.dot(t, l2_ref[...],
                         preferred_element_type=jnp.float32) + b2_ref[...]


def _head_mlp(a, l1, b1row, l2, b2row, bm):
    m, k = a.shape
    h = l1.shape[1]
    nn = l2.shape[1]
    return pl.pallas_call(
        _head_body,
        grid=(m // bm,),
        in_specs=[pl.BlockSpec((bm, k), lambda i: (i, 0)),
                  pl.BlockSpec((k, h), lambda i: (0, 0)),
                  pl.BlockSpec((1, h), lambda i: (0, 0)),
                  pl.BlockSpec((h, nn), lambda i: (0, 0)),
                  pl.BlockSpec((1, nn), lambda i: (0, 0))],
        out_specs=pl.BlockSpec((bm, nn), lambda i: (i, 0)),
        out_shape=jax.ShapeDtypeStruct((m, nn), jnp.float32),
    )(a, l1, b1row, l2, b2row)


def _make_sc_alpha(e, npad, aoff):
    epw = e // (_NC * _NS)
    nsub = 4
    eps = epw // nsub
    nich = eps // _ICH
    assert eps % _ICH == 0
    ngrp = (eps + 15) // 16
    zrows = npad // _NS
    mesh = plsc.VectorSubcoreMesh(core_axis_name="c", subcore_axis_name="s")

    @functools.partial(
        pl.kernel,
        out_type=(jax.ShapeDtypeStruct((e, 16), jnp.float32),
                  jax.ShapeDtypeStruct((_NC, npad, 16), jnp.float32)),
        mesh=mesh,
        compiler_params=_SC_PARAMS,
        scratch_types=[
            pltpu.VMEM((nich, _ICH), jnp.int32),
            pltpu.VMEM((nich, _ICH), jnp.int32),
            pltpu.VMEM((eps, 16), jnp.float32),
            pltpu.VMEM((eps, 16), jnp.float32),
            pltpu.VMEM((eps, 16), jnp.float32),
            pltpu.VMEM((eps, 16), jnp.float32),
            pltpu.SemaphoreType.DMA,
            pltpu.SemaphoreType.DMA,
            pltpu.VMEM_SHARED((npad, 16), jnp.float32),
        ],
    )
    def sc_alpha(src2_hbm, dst2_hbm, avd_hbm, ae_hbm, z16_hbm,
                 expal_hbm, dnm_hbm, srcv, dstv, rs, rd, ra, ex, sem, sem2,
                 dshared):
        c = lax.axis_index("c")
        s = lax.axis_index("s")
        w = c * _NS + s
        pltpu.sync_copy(z16_hbm.at[pl.ds(s * zrows, zrows)],
                        dshared.at[pl.ds(s * zrows, zrows)])
        iota = lax.iota(jnp.int32, 16)
        zero16 = (iota * 0).astype(jnp.float32)

        def zex(g, _):
            ex[g, :] = zero16
            return 0

        lax.fori_loop(0, eps, zex, 0)
        plsc.subcore_barrier()

        for sub in range(nsub):
            base = w * epw + sub * eps
            rbase = base // _ICH
            pltpu.sync_copy(src2_hbm.at[pl.ds(rbase, nich)], srcv)
            pltpu.sync_copy(dst2_hbm.at[pl.ds(rbase, nich)], dstv)
            pltpu.sync_copy(ae_hbm.at[pl.ds(base, eps)], ra)
            descs = []
            for i in range(nich):
                descs.append(pltpu.async_copy(
                    avd_hbm.at[srcv.at[i]],
                    rs.at[pl.ds(i * _ICH, _ICH)], sem))
                descs.append(pltpu.async_copy(
                    avd_hbm.at[dstv.at[i]],
                    rd.at[pl.ds(i * _ICH, _ICH)], sem))
                if len(descs) >= 8:
                    for dsc in descs:
                        dsc.wait()
                    descs = []
            for dsc in descs:
                dsc.wait()

            def grp_body(g, _):
                e16 = g * 16 + iota
                msk = e16 < eps
                for h in range(4):
                    a1 = plsc.load_gather(rs, [e16, iota * 0 + h], mask=msk)
                    a2 = plsc.load_gather(rd, [e16, iota * 0 + (4 + h)],
                                          mask=msk)
                    a3 = plsc.load_gather(ra, [e16, iota * 0 + (aoff + h)],
                                          mask=msk)
                    al = a1 + a2 + a3
                    al = jnp.maximum(al, al * 0.2)
                    exv = jnp.exp(al)
                    plsc.store_scatter(ex, [e16, iota * 0 + h], exv,
                                       mask=msk)
                return 0

            lax.fori_loop(0, ngrp, grp_body, 0)
            pltpu.sync_copy(ex, expal_hbm.at[pl.ds(base, eps)])
            descs = []
            for i in range(nich):
                descs.append(pltpu.async_copy(
                    ex.at[pl.ds(i * _ICH, _ICH)], dshared.at[dstv.at[i]],
                    sem2, add=True))
                if len(descs) >= 8:
                    for dsc in descs:
                        dsc.wait()
                    descs = []
            for dsc in descs:
                dsc.wait()
        plsc.subcore_barrier()
        pltpu.sync_copy(dshared.at[pl.ds(s * zrows, zrows)],
                        dnm_hbm.at[c, pl.ds(s * zrows, zrows)])

    return sc_alpha


def _make_sc_agg(e, npad, concat):
    ept = e // _NS
    nchk = ept // _CH
    assert ept % _CH == 0 and _CH % 16 == 0
    ngrp = _CH // 16
    rt = _R // _NS
    outw = 1024 if concat else 256
    mesh = plsc.VectorSubcoreMesh(core_axis_name="c", subcore_axis_name="s")

    @functools.partial(
        pl.kernel,
        out_type=jax.ShapeDtypeStruct((npad, outw), jnp.float32),
        mesh=mesh,
        compiler_params=_SC_PARAMS,
        scratch_types=[
            pltpu.VMEM((_CH,), jnp.int32),
            pltpu.VMEM((_CH,), jnp.int32),
            pltpu.VMEM((_CH, 16), jnp.float32),
            pltpu.VMEM((rt * 16,), jnp.float32),
            pltpu.VMEM((rt * 16,), jnp.float32),
            pltpu.VMEM((rt * 16,), jnp.float32),
            pltpu.VMEM((_CH + 16,), jnp.int32),
            pltpu.VMEM((_CH + 16,), jnp.int32),
            pltpu.VMEM((_CH + 16,), jnp.float32),
            pltpu.VMEM((_CH + 16,), jnp.float32),
            pltpu.VMEM((_CH + 16,), jnp.float32),
            pltpu.VMEM((_CH + 16,), jnp.float32),
            pltpu.VMEM((16, 1024), jnp.float32),
            pltpu.VMEM((16, 256), jnp.float32),
            pltpu.SemaphoreType.DMA,
            pltpu.VMEM_SHARED((_R + 8, 1024), jnp.float32),
        ],
    )
    def sc_agg(src_hbm, dst_hbm, ea_hbm, dnm_hbm, xp_hbm, zacc_hbm, out_hbm,
               srcv, dstv, eav, d0v, d1v, invv, csrc, cdst, cw0, cw1, cw2,
               cw3, rowbuf, outbuf, sem, accum):
        cw = (cw0, cw1, cw2, cw3)
        c = lax.axis_index("c")
        s = lax.axis_index("s")
        iota = lax.iota(jnp.int32, 16)
        izero16 = iota * 0
        zero16 = izero16.astype(jnp.float32)
        qtr = zero16 + 0.25

        def pass_body(p, _):
            cid = c * _NPASS + p
            lo = cid * _R
            pltpu.sync_copy(zacc_hbm.at[pl.ds(s * rt, rt)],
                            accum.at[pl.ds(s * rt, rt)])
            dbase = (lo + s * rt) * 16
            pltpu.sync_copy(dnm_hbm.at[0, pl.ds(dbase, rt * 16)], d0v)
            pltpu.sync_copy(dnm_hbm.at[1, pl.ds(dbase, rt * 16)], d1v)

            def inv_body(g, _):
                sl = pl.ds(g * 16, 16)
                invv[sl] = 1.0 / (d0v[sl] + d1v[sl] + _EPS)
                return 0

            lax.fori_loop(0, rt, inv_body, 0)
            plsc.subcore_barrier()

            def chunk_body(k, _):
                ebase = s * ept + k * _CH
                pltpu.sync_copy(src_hbm.at[pl.ds(ebase, _CH)], srcv)
                pltpu.sync_copy(dst_hbm.at[pl.ds(ebase, _CH)], dstv)
                pltpu.sync_copy(ea_hbm.at[pl.ds(ebase, _CH)], eav)

                def comp_body(g, cur):
                    sl = pl.ds(g * 16, 16)
                    dv = dstv[sl]
                    sv = srcv[sl]
                    dloc = dv - lo
                    m = (dv >= lo) & (dv < lo + _R)
                    csl = pl.ds(cur, 16)
                    plsc.store_compressed(csrc.at[csl], sv, mask=m)
                    plsc.store_compressed(cdst.at[csl], dloc, mask=m)
                    e16 = g * 16 + iota
                    for h in range(4):
                        eh = plsc.load_gather(eav, [e16, izero16 + h])
                        plsc.store_compressed(cw[h].at[csl], eh, mask=m)
                    cnt = jnp.max(plsc.all_reduce_population_count(m))
                    return cur + cnt

                cur = lax.fori_loop(0, ngrp, comp_body, 0)
                tsl = pl.ds(cur, 16)
                csrc[tsl] = izero16
                cdst[tsl] = izero16 + _R
                for h in range(4):
                    cw[h][tsl] = zero16
                nb = (cur + 15) // 16

                def agg_body(b, _):
                    off = b * 16
                    idxv = csrc[pl.ds(off, 16)]
                    pltpu.async_copy(xp_hbm.at[idxv], rowbuf, sem).wait()

                    def scale_row(j, _):
                        for h in range(4):
                            wv = plsc.load_gather(
                                cw[h], [izero16 + (off + j)])
                            for v in range(16):
                                sl2 = pl.ds(h * 256 + v * 16, 16)
                                rowbuf[j, sl2] = rowbuf[j, sl2] * wv
                        return 0

                    lax.fori_loop(0, 16, scale_row, 0)
                    dv16 = cdst[pl.ds(off, 16)]
                    pltpu.sync_copy(rowbuf, accum.at[dv16], add=True)
                    return 0

                lax.fori_loop(0, nb, agg_body, 0)
                return 0

            lax.fori_loop(0, nchk, chunk_body, 0)
            plsc.subcore_barrier()

            def emit_grp(r0, rl0, gsz):
                pltpu.sync_copy(accum.at[pl.ds(r0, gsz)],
                                rowbuf.at[pl.ds(0, gsz)])

                def fin_row(j, _):
                    for h in range(4):
                        wv = plsc.load_gather(
                            invv, [izero16 + ((rl0 + j) * 16 + h)])
                        if concat:
                            for v in range(16):
                                sl2 = pl.ds(h * 256 + v * 16, 16)
                                rowbuf[j, sl2] = rowbuf[j, sl2] * wv
                        else:
                            for v in range(16):
                                sl2 = pl.ds(h * 256 + v * 16, 16)
                                osl = pl.ds(v * 16, 16)
                                t = rowbuf[j, sl2] * wv * qtr
                                if h == 0:
                                    outbuf[j, osl] = t
                                else:
                                    outbuf[j, osl] = outbuf[j, osl] + t
                    return 0

                lax.fori_loop(0, gsz, fin_row, 0)
                srcbuf = rowbuf if concat else outbuf
                pltpu.sync_copy(srcbuf.at[pl.ds(0, gsz)],
                                out_hbm.at[pl.ds(lo + r0, gsz)])

            def emit16(gi, _):
                emit_grp(s * rt + gi * 16, gi * 16, 16)
                return 0

            lax.fori_loop(0, rt // 16, emit16, 0)
            if rt % 16:
                emit_grp(s * rt + (rt // 16) * 16, (rt // 16) * 16, rt % 16)
            plsc.subcore_barrier()
            return 0

        lax.fori_loop(0, _NPASS, pass_body, 0)

    return sc_agg


def kernel(x, edge_index, edge_attr, W1, as1, ad1, We1, ae1, b1,
           W2, as2, ad2, We2, ae2, b2, L1, bL1, L2, bL2):
    n, din = x.shape
    e, edim = edge_attr.shape
    H, C = as1.shape[1], as1.shape[2]
    f32 = jnp.float32

    src = edge_index[0].astype(jnp.int32)
    dst = edge_index[1].astype(jnp.int32)
    src2 = src.reshape(e // _ICH, _ICH)
    dst2 = dst.reshape(e // _ICH, _ICH)

    vs1 = (W1.reshape(din, H, C) * as1).sum(-1)
    vd1 = (W1.reshape(din, H, C) * ad1).sum(-1)
    vsd1 = jnp.concatenate([vs1, vd1], axis=1)
    vsd1 = jnp.pad(vsd1, ((0, 0), (0, 120)))
    hc = H * C
    vs2 = (W2.reshape(hc, H, C) * as2).sum(-1)
    vd2 = (W2.reshape(hc, H, C) * ad2).sum(-1)
    vsd2 = jnp.concatenate([vs2, vd2], axis=1)
    vsd2 = jnp.pad(vsd2, ((0, 0), (0, 120)))
    me1 = (We1.reshape(edim, H, C) * ae1).sum(-1)
    me2 = (We2.reshape(edim, H, C) * ae2).sum(-1)
    mcat = jnp.pad(jnp.concatenate([me1, me2], axis=1),
                   ((0, 0), (0, 8)))
    wblk = jnp.kron(jnp.eye(8, dtype=f32), mcat)

    zrow1024 = jnp.zeros((1, 1024), f32)
    zrow128 = jnp.zeros((1, 128), f32)
    z16 = jnp.zeros((_NPAD, 16), f32)
    zacc = jnp.zeros((_R, 1024), f32)

    xpad = jnp.concatenate([x, jnp.zeros((_NPAD - n, din), f32)], axis=0)

    eaf = _matmul(edge_attr.reshape(e // 8, 128), wblk, zrow128, bm=2000)
    ea16 = eaf.reshape(e, 16)

    sc_alpha1 = _make_sc_alpha(e, _NPAD, aoff=0)
    sc_alpha2 = _make_sc_alpha(e, _NPAD, aoff=4)
    sc_agg_c = _make_sc_agg(e, _NPAD, concat=True)
    sc_agg_m = _make_sc_agg(e, _NPAD, concat=False)

    xp1 = _matmul(xpad, W1, zrow1024, bm=768)
    ac1 = _matmul(xpad, vsd1, zrow128, bm=768)
    avd1 = ac1[:, 0:16]
    expal1, dnm1 = sc_alpha1(src2, dst2, avd1, ea16, z16)
    h1 = sc_agg_c(src, dst, expal1, dnm1.reshape(_NC, _NPAD * 16), xp1,
                  zacc)

    xp2 = _matmul(h1, W2, (b1 @ W2).reshape(1, hc), bm=768)
    ac2 = _matmul(h1, vsd2, (b1 @ vsd2).reshape(1, 128), bm=768)
    avd2 = ac2[:, 0:16]
    expal2, dnm2 = sc_alpha2(src2, dst2, avd2, ea16, z16)
    h2 = sc_agg_m(src, dst, expal2, dnm2.reshape(_NC, _NPAD * 16), xp2,
                  zacc)

    out = _head_mlp(h2, L1, (b2 @ L1 + bL1).reshape(1, C),
                    L2, bL2.reshape(1, -1), bm=768)
    return out[:n]


# speedup vs baseline: 8.3665x; 1.1138x over previous
import functools

import jax
import jax.numpy as jnp
from jax import lax
from jax.experimental import pallas as pl
from jax.experimental.pallas import tpu as pltpu
from jax.experimental.pallas import tpu_sc as plsc

_NC = 2
_NS = 16
_R = 896
_NPASS = 6
_NPAD = _NC * _NPASS * _R
_CH = 400
_ICH = 125
_EPS = 1e-16
_SC_PARAMS = pltpu.CompilerParams(needs_layout_passes=False,
                                  use_tc_tiling_on_sc=False)


def _mm_body(a_ref, b_ref, bias_ref, o_ref):
    o_ref[...] = jnp.dot(a_ref[...], b_ref[...],
                         preferred_element_type=jnp.float32) + bias_ref[...]


def _matmul(a, b, bias_row, bm):
    m, k = a.shape
    _, nn = b.shape
    grid = (m // bm,)
    return pl.pallas_call(
        _mm_body,
        grid=grid,
        in_specs=[pl.BlockSpec((bm, k), lambda i: (i, 0)),
                  pl.BlockSpec((k, nn), lambda i: (0, 0)),
                  pl.BlockSpec((1, nn), lambda i: (0, 0))],
        out_specs=pl.BlockSpec((bm, nn), lambda i: (i, 0)),
        out_shape=jax.ShapeDtypeStruct((m, nn), jnp.float32),
    )(a, b, bias_row)


def _head_body(a_ref, l1_ref, b1_ref, l2_ref, b2_ref, o_ref):
    t = jnp.dot(a_ref[...], l1_ref[...],
                preferred_element_type=jnp.float32) + b1_ref[...]
    t = jnp.maximum(t, 0.0)
    o_ref[...] = jnp.dot(t, l2_ref[...],
                         preferred_element_type=jnp.float32) + b2_ref[...]


def _head_mlp(a, l1, b1row, l2, b2row, bm):
    m, k = a.shape
    h = l1.shape[1]
    nn = l2.shape[1]
    return pl.pallas_call(
        _head_body,
        grid=(m // bm,),
        in_specs=[pl.BlockSpec((bm, k), lambda i: (i, 0)),
                  pl.BlockSpec((k, h), lambda i: (0, 0)),
                  pl.BlockSpec((1, h), lambda i: (0, 0)),
                  pl.BlockSpec((h, nn), lambda i: (0, 0)),
                  pl.BlockSpec((1, nn), lambda i: (0, 0))],
        out_specs=pl.BlockSpec((bm, nn), lambda i: (i, 0)),
        out_shape=jax.ShapeDtypeStruct((m, nn), jnp.float32),
    )(a, l1, b1row, l2, b2row)


def _make_sc_alpha(e, npad, aoff):
    epw = e // (_NC * _NS)
    nsub = 4
    eps = epw // nsub
    nich = eps // _ICH
    assert eps % _ICH == 0
    ngrp = (eps + 15) // 16
    zrows = npad // _NS
    mesh = plsc.VectorSubcoreMesh(core_axis_name="c", subcore_axis_name="s")

    @functools.partial(
        pl.kernel,
        out_type=(jax.ShapeDtypeStruct((e, 16), jnp.float32),
                  jax.ShapeDtypeStruct((_NC, npad, 16), jnp.float32)),
        mesh=mesh,
        compiler_params=_SC_PARAMS,
        scratch_types=[
            pltpu.VMEM((nich, _ICH), jnp.int32),
            pltpu.VMEM((nich, _ICH), jnp.int32),
            pltpu.VMEM((eps, 16), jnp.float32),
            pltpu.VMEM((eps, 16), jnp.float32),
            pltpu.VMEM((eps, 16), jnp.float32),
            pltpu.VMEM((eps, 16), jnp.float32),
            pltpu.SemaphoreType.DMA,
            pltpu.SemaphoreType.DMA,
            pltpu.VMEM_SHARED((npad, 16), jnp.float32),
        ],
    )
    def sc_alpha(src2_hbm, dst2_hbm, avd_hbm, ae_hbm, z16_hbm,
                 expal_hbm, dnm_hbm, srcv, dstv, rs, rd, ra, ex, sem, sem2,
                 dshared):
        c = lax.axis_index("c")
        s = lax.axis_index("s")
        w = c * _NS + s
        pltpu.sync_copy(z16_hbm.at[pl.ds(s * zrows, zrows)],
                        dshared.at[pl.ds(s * zrows, zrows)])
        iota = lax.iota(jnp.int32, 16)
        zero16 = (iota * 0).astype(jnp.float32)

        def zex(g, _):
            ex[g, :] = zero16
            return 0

        lax.fori_loop(0, eps, zex, 0)
        plsc.subcore_barrier()

        for sub in range(nsub):
            base = w * epw + sub * eps
            rbase = base // _ICH
            pltpu.sync_copy(src2_hbm.at[pl.ds(rbase, nich)], srcv)
            pltpu.sync_copy(dst2_hbm.at[pl.ds(rbase, nich)], dstv)
            pltpu.sync_copy(ae_hbm.at[pl.ds(base, eps)], ra)
            descs = []
            for i in range(nich):
                descs.append(pltpu.async_copy(
                    avd_hbm.at[srcv.at[i]],
                    rs.at[pl.ds(i * _ICH, _ICH)], sem))
                descs.append(pltpu.async_copy(
                    avd_hbm.at[dstv.at[i]],
                    rd.at[pl.ds(i * _ICH, _ICH)], sem))
                if len(descs) >= 8:
                    for dsc in descs:
                        dsc.wait()
                    descs = []
            for dsc in descs:
                dsc.wait()

            def grp_body(g, _):
                e16 = g * 16 + iota
                msk = e16 < eps
                for h in range(4):
                    a1 = plsc.load_gather(rs, [e16, iota * 0 + h], mask=msk)
                    a2 = plsc.load_gather(rd, [e16, iota * 0 + (4 + h)],
                                          mask=msk)
                    a3 = plsc.load_gather(ra, [e16, iota * 0 + (aoff + h)],
                                          mask=msk)
                    al = a1 + a2 + a3
                    al = jnp.maximum(al, al * 0.2)
                    exv = jnp.exp(al)
                    plsc.store_scatter(ex, [e16, iota * 0 + h], exv,
                                       mask=msk)
                return 0

            lax.fori_loop(0, ngrp, grp_body, 0)
            pltpu.sync_copy(ex, expal_hbm.at[pl.ds(base, eps)])
            descs = []
            for i in range(nich):
                descs.append(pltpu.async_copy(
                    ex.at[pl.ds(i * _ICH, _ICH)], dshared.at[dstv.at[i]],
                    sem2, add=True))
                if len(descs) >= 8:
                    for dsc in descs:
                        dsc.wait()
                    descs = []
            for dsc in descs:
                dsc.wait()
        plsc.subcore_barrier()
        pltpu.sync_copy(dshared.at[pl.ds(s * zrows, zrows)],
                        dnm_hbm.at[c, pl.ds(s * zrows, zrows)])

    return sc_alpha


def _make_sc_agg(e, npad, concat):
    ept = e // _NS
    nchk = ept // _CH
    assert ept % _CH == 0 and _CH % 16 == 0
    ngrp = _CH // 16
    rt = _R // _NS
    outw = 1024 if concat else 256
    mesh = plsc.VectorSubcoreMesh(core_axis_name="c", subcore_axis_name="s")

    @functools.partial(
        pl.kernel,
        out_type=jax.ShapeDtypeStruct((npad, outw), jnp.float32),
        mesh=mesh,
        compiler_params=_SC_PARAMS,
        scratch_types=[
            pltpu.VMEM((_CH,), jnp.int32),
            pltpu.VMEM((_CH,), jnp.int32),
            pltpu.VMEM((_CH, 16), jnp.float32),
            pltpu.VMEM((rt * 16,), jnp.float32),
            pltpu.VMEM((rt * 16,), jnp.float32),
            pltpu.VMEM((rt * 16,), jnp.float32),
            pltpu.VMEM((_CH + 16,), jnp.int32),
            pltpu.VMEM((_CH + 16,), jnp.int32),
            pltpu.VMEM((_CH + 16,), jnp.float32),
            pltpu.VMEM((_CH + 16,), jnp.float32),
            pltpu.VMEM((_CH + 16,), jnp.float32),
            pltpu.VMEM((_CH + 16,), jnp.float32),
            pltpu.VMEM((16, 1024), jnp.float32),
            pltpu.VMEM((16, 1024), jnp.float32),
            pltpu.VMEM((16, 256), jnp.float32),
            pltpu.SemaphoreType.DMA,
            pltpu.SemaphoreType.DMA,
            pltpu.VMEM_SHARED((_R + 8, 1024), jnp.float32),
        ],
    )
    def sc_agg(src_hbm, dst_hbm, ea_hbm, dnm_hbm, xp_hbm, zacc_hbm, out_hbm,
               srcv, dstv, eav, d0v, d1v, invv, csrc, cdst, cw0, cw1, cw2,
               cw3, rowbuf, rowbuf2, outbuf, sem, sem2, accum):
        cw = (cw0, cw1, cw2, cw3)
        c = lax.axis_index("c")
        s = lax.axis_index("s")
        iota = lax.iota(jnp.int32, 16)
        izero16 = iota * 0
        zero16 = izero16.astype(jnp.float32)
        qtr = zero16 + 0.25

        def pass_body(p, _):
            cid = c * _NPASS + p
            lo = cid * _R
            pltpu.sync_copy(zacc_hbm.at[pl.ds(s * rt, rt)],
                            accum.at[pl.ds(s * rt, rt)])
            dbase = (lo + s * rt) * 16
            pltpu.sync_copy(dnm_hbm.at[0, pl.ds(dbase, rt * 16)], d0v)
            pltpu.sync_copy(dnm_hbm.at[1, pl.ds(dbase, rt * 16)], d1v)

            def inv_body(g, _):
                sl = pl.ds(g * 16, 16)
                invv[sl] = 1.0 / (d0v[sl] + d1v[sl] + _EPS)
                return 0

            lax.fori_loop(0, rt, inv_body, 0)
            plsc.subcore_barrier()

            def chunk_body(k, _):
                ebase = s * ept + k * _CH
                pltpu.sync_copy(src_hbm.at[pl.ds(ebase, _CH)], srcv)
                pltpu.sync_copy(dst_hbm.at[pl.ds(ebase, _CH)], dstv)
                pltpu.sync_copy(ea_hbm.at[pl.ds(ebase, _CH)], eav)

                def comp_body(g, cur):
                    sl = pl.ds(g * 16, 16)
                    dv = dstv[sl]
                    sv = srcv[sl]
                    dloc = dv - lo
                    m = (dv >= lo) & (dv < lo + _R)
                    csl = pl.ds(cur, 16)
                    plsc.store_compressed(csrc.at[csl], sv, mask=m)
                    plsc.store_compressed(cdst.at[csl], dloc, mask=m)
                    e16 = g * 16 + iota
                    for h in range(4):
                        eh = plsc.load_gather(eav, [e16, izero16 + h])
                        plsc.store_compressed(cw[h].at[csl], eh, mask=m)
                    cnt = jnp.max(plsc.all_reduce_population_count(m))
                    return cur + cnt

                cur = lax.fori_loop(0, ngrp, comp_body, 0)
                tsl = pl.ds(cur, 16)
                csrc[tsl] = izero16
                cdst[tsl] = izero16 + _R
                for h in range(4):
                    cw[h][tsl] = zero16
                nb = (cur + 15) // 16

                @pl.when(nb > 0)
                def _():
                    pltpu.async_copy(xp_hbm.at[csrc[pl.ds(0, 16)]],
                                     rowbuf, sem)

                bufs = ((rowbuf, sem), (rowbuf2, sem2))

                def slot(b, rb, gsem, rbo, gsemo):
                    off = b * 16

                    @pl.when(b + 1 < nb)
                    def _():
                        idxn = csrc[pl.ds(off + 16, 16)]
                        pltpu.async_copy(xp_hbm.at[idxn], rbo, gsemo)

                    idxv = csrc[pl.ds(off, 16)]
                    pltpu.make_async_copy(xp_hbm.at[idxv], rb, gsem).wait()

                    def scale_row(j, _):
                        for h in range(4):
                            wv = plsc.load_gather(
                                cw[h], [izero16 + (off + j)])
                            for v in range(16):
                                sl2 = pl.ds(h * 256 + v * 16, 16)
                                rb[j, sl2] = rb[j, sl2] * wv
                        return 0

                    lax.fori_loop(0, 16, scale_row, 0)
                    dv16 = cdst[pl.ds(off, 16)]
                    pltpu.sync_copy(rb, accum.at[dv16], add=True)

                def pair_body(q, _):
                    for par in (0, 1):
                        b = q * 2 + par
                        rb, gsem = bufs[par]
                        rbo, gsemo = bufs[1 - par]

                        @pl.when(b < nb)
                        def _():
                            slot(b, rb, gsem, rbo, gsemo)
                    return 0

                lax.fori_loop(0, (nb + 1) // 2, pair_body, 0)
                return 0

            lax.fori_loop(0, nchk, chunk_body, 0)
            plsc.subcore_barrier()

            def emit_grp(r0, rl0, gsz):
                pltpu.sync_copy(accum.at[pl.ds(r0, gsz)],
                                rowbuf.at[pl.ds(0, gsz)])

                def fin_row(j, _):
                    for h in range(4):
                        wv = plsc.load_gather(
                            invv, [izero16 + ((rl0 + j) * 16 + h)])
                        if concat:
                            for v in range(16):
                                sl2 = pl.ds(h * 256 + v * 16, 16)
                                rowbuf[j, sl2] = rowbuf[j, sl2] * wv
                        else:
                            for v in range(16):
                                sl2 = pl.ds(h * 256 + v * 16, 16)
                                osl = pl.ds(v * 16, 16)
                                t = rowbuf[j, sl2] * wv * qtr
                                if h == 0:
                                    outbuf[j, osl] = t
                                else:
                                    outbuf[j, osl] = outbuf[j, osl] + t
                    return 0

                lax.fori_loop(0, gsz, fin_row, 0)
                srcbuf = rowbuf if concat else outbuf
                pltpu.sync_copy(srcbuf.at[pl.ds(0, gsz)],
                                out_hbm.at[pl.ds(lo + r0, gsz)])

            def emit16(gi, _):
                emit_grp(s * rt + gi * 16, gi * 16, 16)
                return 0

            lax.fori_loop(0, rt // 16, emit16, 0)
            if rt % 16:
                emit_grp(s * rt + (rt // 16) * 16, (rt // 16) * 16, rt % 16)
            plsc.subcore_barrier()
            return 0

        lax.fori_loop(0, _NPASS, pass_body, 0)

    return sc_agg


def kernel(x, edge_index, edge_attr, W1, as1, ad1, We1, ae1, b1,
           W2, as2, ad2, We2, ae2, b2, L1, bL1, L2, bL2):
    n, din = x.shape
    e, edim = edge_attr.shape
    H, C = as1.shape[1], as1.shape[2]
    f32 = jnp.float32

    src = edge_index[0].astype(jnp.int32)
    dst = edge_index[1].astype(jnp.int32)
    src2 = src.reshape(e // _ICH, _ICH)
    dst2 = dst.reshape(e // _ICH, _ICH)

    vs1 = (W1.reshape(din, H, C) * as1).sum(-1)
    vd1 = (W1.reshape(din, H, C) * ad1).sum(-1)
    vsd1 = jnp.concatenate([vs1, vd1], axis=1)
    vsd1 = jnp.pad(vsd1, ((0, 0), (0, 120)))
    hc = H * C
    vs2 = (W2.reshape(hc, H, C) * as2).sum(-1)
    vd2 = (W2.reshape(hc, H, C) * ad2).sum(-1)
    vsd2 = jnp.concatenate([vs2, vd2], axis=1)
    vsd2 = jnp.pad(vsd2, ((0, 0), (0, 120)))
    me1 = (We1.reshape(edim, H, C) * ae1).sum(-1)
    me2 = (We2.reshape(edim, H, C) * ae2).sum(-1)
    mcat = jnp.pad(jnp.concatenate([me1, me2], axis=1),
                   ((0, 0), (0, 8)))
    wblk = jnp.kron(jnp.eye(8, dtype=f32), mcat)

    zrow1024 = jnp.zeros((1, 1024), f32)
    zrow128 = jnp.zeros((1, 128), f32)
    z16 = jnp.zeros((_NPAD, 16), f32)
    zacc = jnp.zeros((_R, 1024), f32)

    xpad = jnp.concatenate([x, jnp.zeros((_NPAD - n, din), f32)], axis=0)

    eaf = _matmul(edge_attr.reshape(e // 8, 128), wblk, zrow128, bm=2000)
    ea16 = eaf.reshape(e, 16)

    sc_alpha1 = _make_sc_alpha(e, _NPAD, aoff=0)
    sc_alpha2 = _make_sc_alpha(e, _NPAD, aoff=4)
    sc_agg_c = _make_sc_agg(e, _NPAD, concat=True)
    sc_agg_m = _make_sc_agg(e, _NPAD, concat=False)

    xp1 = _matmul(xpad, W1, zrow1024, bm=768)
    ac1 = _matmul(xpad, vsd1, zrow128, bm=768)
    avd1 = ac1[:, 0:16]
    expal1, dnm1 = sc_alpha1(src2, dst2, avd1, ea16, z16)
    h1 = sc_agg_c(src, dst, expal1, dnm1.reshape(_NC, _NPAD * 16), xp1,
                  zacc)

    xp2 = _matmul(h1, W2, (b1 @ W2).reshape(1, hc), bm=768)
    ac2 = _matmul(h1, vsd2, (b1 @ vsd2).reshape(1, 128), bm=768)
    avd2 = ac2[:, 0:16]
    expal2, dnm2 = sc_alpha2(src2, dst2, avd2, ea16, z16)
    h2 = sc_agg_m(src, dst, expal2, dnm2.reshape(_NC, _NPAD * 16), xp2,
                  zacc)

    out = _head_mlp(h2, L1, (b2 @ L1 + bL1).reshape(1, C),
                    L2, bL2.reshape(1, -1), bm=768)
    return out[:n]


# speedup vs baseline: 8.7234x; 1.0427x over previous
import functools

import jax
import jax.numpy as jnp
from jax import lax
from jax.experimental import pallas as pl
from jax.experimental.pallas import tpu as pltpu
from jax.experimental.pallas import tpu_sc as plsc

_NC = 2
_NS = 16
_R = 896
_NPASS = 6
_NPAD = _NC * _NPASS * _R
_CH = 400
_ICH = 125
_EPS = 1e-16
_SC_PARAMS = pltpu.CompilerParams(needs_layout_passes=False,
                                  use_tc_tiling_on_sc=False)


def _mm_body(a_ref, b_ref, bias_ref, o_ref):
    o_ref[...] = jnp.dot(a_ref[...], b_ref[...],
                         preferred_element_type=jnp.float32) + bias_ref[...]


def _matmul(a, b, bias_row, bm):
    m, k = a.shape
    _, nn = b.shape
    grid = (m // bm,)
    return pl.pallas_call(
        _mm_body,
        grid=grid,
        in_specs=[pl.BlockSpec((bm, k), lambda i: (i, 0)),
                  pl.BlockSpec((k, nn), lambda i: (0, 0)),
                  pl.BlockSpec((1, nn), lambda i: (0, 0))],
        out_specs=pl.BlockSpec((bm, nn), lambda i: (i, 0)),
        out_shape=jax.ShapeDtypeStruct((m, nn), jnp.float32),
    )(a, b, bias_row)


def _head_body(a_ref, l1_ref, b1_ref, l2_ref, b2_ref, o_ref):
    t = jnp.dot(a_ref[...], l1_ref[...],
                preferred_element_type=jnp.float32) + b1_ref[...]
    t = jnp.maximum(t, 0.0)
    o_ref[...] = jnp.dot(t, l2_ref[...],
                         preferred_element_type=jnp.float32) + b2_ref[...]


def _head_mlp(a, l1, b1row, l2, b2row, bm):
    m, k = a.shape
    h = l1.shape[1]
    nn = l2.shape[1]
    return pl.pallas_call(
        _head_body,
        grid=(m // bm,),
        in_specs=[pl.BlockSpec((bm, k), lambda i: (i, 0)),
                  pl.BlockSpec((k, h), lambda i: (0, 0)),
                  pl.BlockSpec((1, h), lambda i: (0, 0)),
                  pl.BlockSpec((h, nn), lambda i: (0, 0)),
                  pl.BlockSpec((1, nn), lambda i: (0, 0))],
        out_specs=pl.BlockSpec((bm, nn), lambda i: (i, 0)),
        out_shape=jax.ShapeDtypeStruct((m, nn), jnp.float32),
    )(a, l1, b1row, l2, b2row)


def _make_sc_alpha(e, npad, aoff):
    epw = e // (_NC * _NS)
    nsub = 4
    eps = epw // nsub
    nich = eps // _ICH
    assert eps % _ICH == 0
    ngrp = (eps + 15) // 16
    zrows = npad // _NS
    mesh = plsc.VectorSubcoreMesh(core_axis_name="c", subcore_axis_name="s")

    @functools.partial(
        pl.kernel,
        out_type=(jax.ShapeDtypeStruct((e, 16), jnp.float32),
                  jax.ShapeDtypeStruct((_NC, npad, 16), jnp.float32)),
        mesh=mesh,
        compiler_params=_SC_PARAMS,
        scratch_types=[
            pltpu.VMEM((nich, _ICH), jnp.int32),
            pltpu.VMEM((nich, _ICH), jnp.int32),
            pltpu.VMEM((eps, 16), jnp.float32),
            pltpu.VMEM((eps, 16), jnp.float32),
            pltpu.VMEM((eps, 16), jnp.float32),
            pltpu.VMEM((eps, 16), jnp.float32),
            pltpu.SemaphoreType.DMA,
            pltpu.SemaphoreType.DMA,
            pltpu.VMEM_SHARED((npad, 16), jnp.float32),
        ],
    )
    def sc_alpha(src2_hbm, dst2_hbm, avd_hbm, ae_hbm, z16_hbm,
                 expal_hbm, dnm_hbm, srcv, dstv, rs, rd, ra, ex, sem, sem2,
                 dshared):
        c = lax.axis_index("c")
        s = lax.axis_index("s")
        w = c * _NS + s
        pltpu.sync_copy(z16_hbm.at[pl.ds(s * zrows, zrows)],
                        dshared.at[pl.ds(s * zrows, zrows)])
        iota = lax.iota(jnp.int32, 16)
        zero16 = (iota * 0).astype(jnp.float32)

        def zex(g, _):
            ex[g, :] = zero16
            return 0

        lax.fori_loop(0, eps, zex, 0)
        plsc.subcore_barrier()

        for sub in range(nsub):
            base = w * epw + sub * eps
            rbase = base // _ICH
            pltpu.sync_copy(src2_hbm.at[pl.ds(rbase, nich)], srcv)
            pltpu.sync_copy(dst2_hbm.at[pl.ds(rbase, nich)], dstv)
            pltpu.sync_copy(ae_hbm.at[pl.ds(base, eps)], ra)
            descs = []
            for i in range(nich):
                descs.append(pltpu.async_copy(
                    avd_hbm.at[srcv.at[i]],
                    rs.at[pl.ds(i * _ICH, _ICH)], sem))
                descs.append(pltpu.async_copy(
                    avd_hbm.at[dstv.at[i]],
                    rd.at[pl.ds(i * _ICH, _ICH)], sem))
                if len(descs) >= 8:
                    for dsc in descs:
                        dsc.wait()
                    descs = []
            for dsc in descs:
                dsc.wait()

            def grp_body(g, _):
                e16 = g * 16 + iota
                msk = e16 < eps
                for h in range(4):
                    a1 = plsc.load_gather(rs, [e16, iota * 0 + h], mask=msk)
                    a2 = plsc.load_gather(rd, [e16, iota * 0 + (4 + h)],
                                          mask=msk)
                    a3 = plsc.load_gather(ra, [e16, iota * 0 + (aoff + h)],
                                          mask=msk)
                    al = a1 + a2 + a3
                    al = jnp.maximum(al, al * 0.2)
                    exv = jnp.exp(al)
                    plsc.store_scatter(ex, [e16, iota * 0 + h], exv,
                                       mask=msk)
                return 0

            lax.fori_loop(0, ngrp, grp_body, 0)
            pltpu.sync_copy(ex, expal_hbm.at[pl.ds(base, eps)])
            descs = []
            for i in range(nich):
                descs.append(pltpu.async_copy(
                    ex.at[pl.ds(i * _ICH, _ICH)], dshared.at[dstv.at[i]],
                    sem2, add=True))
                if len(descs) >= 8:
                    for dsc in descs:
                        dsc.wait()
                    descs = []
            for dsc in descs:
                dsc.wait()
        plsc.subcore_barrier()
        pltpu.sync_copy(dshared.at[pl.ds(s * zrows, zrows)],
                        dnm_hbm.at[c, pl.ds(s * zrows, zrows)])

    return sc_alpha


def _make_sc_agg(e, npad, concat):
    ept = e // _NS
    nchk = ept // _CH
    assert ept % _CH == 0 and _CH % 16 == 0
    ngrp = _CH // 16
    rt = _R // _NS
    outw = 1024 if concat else 256
    dn_sz = rt * 16 if concat else _R * 16
    inv_sz = rt * 16 if concat else _R * 4
    mesh = plsc.VectorSubcoreMesh(core_axis_name="c", subcore_axis_name="s")

    @functools.partial(
        pl.kernel,
        out_type=jax.ShapeDtypeStruct((npad, outw), jnp.float32),
        mesh=mesh,
        compiler_params=_SC_PARAMS,
        scratch_types=[
            pltpu.VMEM((_CH,), jnp.int32),
            pltpu.VMEM((_CH,), jnp.int32),
            pltpu.VMEM((_CH, 16), jnp.float32),
            pltpu.VMEM((dn_sz,), jnp.float32),
            pltpu.VMEM((dn_sz,), jnp.float32),
            pltpu.VMEM((inv_sz,), jnp.float32),
            pltpu.VMEM((_CH + 16,), jnp.int32),
            pltpu.VMEM((_CH + 16,), jnp.int32),
            pltpu.VMEM((_CH + 16,), jnp.float32),
            pltpu.VMEM((_CH + 16,), jnp.float32),
            pltpu.VMEM((_CH + 16,), jnp.float32),
            pltpu.VMEM((_CH + 16,), jnp.float32),
            pltpu.VMEM((16, 1024), jnp.float32),
            pltpu.VMEM((16, 1024), jnp.float32),
            pltpu.VMEM((16, 256), jnp.float32),
            pltpu.SemaphoreType.DMA,
            pltpu.SemaphoreType.DMA,
            pltpu.VMEM_SHARED((_R + 8, outw), jnp.float32),
        ],
    )
    def sc_agg(src_hbm, dst_hbm, ea_hbm, dnm_hbm, xp_hbm, zacc_hbm, out_hbm,
               srcv, dstv, eav, d0v, d1v, invv, csrc, cdst, cw0, cw1, cw2,
               cw3, rowbuf, rowbuf2, outbuf, sem, sem2, accum):
        cw = (cw0, cw1, cw2, cw3)
        c = lax.axis_index("c")
        s = lax.axis_index("s")
        iota = lax.iota(jnp.int32, 16)
        izero16 = iota * 0
        zero16 = izero16.astype(jnp.float32)
        qtr = zero16 + 0.25

        def pass_body(p, _):
            cid = c * _NPASS + p
            lo = cid * _R
            pltpu.sync_copy(zacc_hbm.at[pl.ds(s * rt, rt)],
                            accum.at[pl.ds(s * rt, rt)])
            if concat:
                dbase = (lo + s * rt) * 16
                pltpu.sync_copy(dnm_hbm.at[0, pl.ds(dbase, rt * 16)], d0v)
                pltpu.sync_copy(dnm_hbm.at[1, pl.ds(dbase, rt * 16)], d1v)

                def inv_body(g, _):
                    sl = pl.ds(g * 16, 16)
                    invv[sl] = 1.0 / (d0v[sl] + d1v[sl] + _EPS)
                    return 0

                lax.fori_loop(0, rt, inv_body, 0)
            else:
                pltpu.sync_copy(dnm_hbm.at[0, pl.ds(lo * 16, _R * 16)], d0v)
                pltpu.sync_copy(dnm_hbm.at[1, pl.ds(lo * 16, _R * 16)], d1v)
                lane = iota

                def inv_body(g, _):
                    fl = g * 16 + lane
                    fidx = lax.shift_left(
                        lax.shift_right_logical(fl, 2), 4) \
                        + lax.bitwise_and(fl, 3)
                    a0 = plsc.load_gather(d0v, [fidx])
                    a1 = plsc.load_gather(d1v, [fidx])
                    invv[pl.ds(g * 16, 16)] = 0.25 / (a0 + a1 + _EPS)
                    return 0

                lax.fori_loop(0, (_R * 4) // 16, inv_body, 0)
            plsc.subcore_barrier()

            def chunk_body(k, _):
                ebase = s * ept + k * _CH
                pltpu.sync_copy(src_hbm.at[pl.ds(ebase, _CH)], srcv)
                pltpu.sync_copy(dst_hbm.at[pl.ds(ebase, _CH)], dstv)
                pltpu.sync_copy(ea_hbm.at[pl.ds(ebase, _CH)], eav)

                def comp_body(g, cur):
                    sl = pl.ds(g * 16, 16)
                    dv = dstv[sl]
                    sv = srcv[sl]
                    dloc = dv - lo
                    m = (dv >= lo) & (dv < lo + _R)
                    csl = pl.ds(cur, 16)
                    plsc.store_compressed(csrc.at[csl], sv, mask=m)
                    plsc.store_compressed(cdst.at[csl], dloc, mask=m)
                    e16 = g * 16 + iota
                    for h in range(4):
                        eh = plsc.load_gather(eav, [e16, izero16 + h])
                        if not concat:
                            ih = plsc.load_gather(
                                invv, [dloc * 4 + h], mask=m)
                            eh = eh * ih
                        plsc.store_compressed(cw[h].at[csl], eh, mask=m)
                    cnt = jnp.max(plsc.all_reduce_population_count(m))
                    return cur + cnt

                cur = lax.fori_loop(0, ngrp, comp_body, 0)
                tsl = pl.ds(cur, 16)
                csrc[tsl] = izero16
                cdst[tsl] = izero16 + _R
                for h in range(4):
                    cw[h][tsl] = zero16
                nb = (cur + 15) // 16

                @pl.when(nb > 0)
                def _():
                    pltpu.async_copy(xp_hbm.at[csrc[pl.ds(0, 16)]],
                                     rowbuf, sem)

                bufs = ((rowbuf, sem), (rowbuf2, sem2))

                def slot(b, rb, gsem, rbo, gsemo):
                    off = b * 16

                    @pl.when(b + 1 < nb)
                    def _():
                        idxn = csrc[pl.ds(off + 16, 16)]
                        pltpu.async_copy(xp_hbm.at[idxn], rbo, gsemo)

                    idxv = csrc[pl.ds(off, 16)]
                    pltpu.make_async_copy(xp_hbm.at[idxv], rb, gsem).wait()

                    if concat:
                        def scale_row(j, _):
                            for h in range(4):
                                wv = plsc.load_gather(
                                    cw[h], [izero16 + (off + j)])
                                for v in range(16):
                                    sl2 = pl.ds(h * 256 + v * 16, 16)
                                    rb[j, sl2] = rb[j, sl2] * wv
                            return 0

                        lax.fori_loop(0, 16, scale_row, 0)
                        dv16 = cdst[pl.ds(off, 16)]
                        pltpu.sync_copy(rb, accum.at[dv16], add=True)
                    else:
                        def scale_row(j, _):
                            ws = [plsc.load_gather(
                                cw[h], [izero16 + (off + j)])
                                for h in range(4)]
                            for v in range(16):
                                t = rb[j, pl.ds(v * 16, 16)] * ws[0]
                                for h in range(1, 4):
                                    sl2 = pl.ds(h * 256 + v * 16, 16)
                                    t = t + rb[j, sl2] * ws[h]
                                outbuf[j, pl.ds(v * 16, 16)] = t
                            return 0

                        lax.fori_loop(0, 16, scale_row, 0)
                        dv16 = cdst[pl.ds(off, 16)]
                        pltpu.sync_copy(outbuf, accum.at[dv16], add=True)

                def pair_body(q, _):
                    for par in (0, 1):
                        b = q * 2 + par
                        rb, gsem = bufs[par]
                        rbo, gsemo = bufs[1 - par]

                        @pl.when(b < nb)
                        def _():
                            slot(b, rb, gsem, rbo, gsemo)
                    return 0

                lax.fori_loop(0, (nb + 1) // 2, pair_body, 0)
                return 0

            lax.fori_loop(0, nchk, chunk_body, 0)
            plsc.subcore_barrier()

            if concat:
                def emit_grp(r0, rl0, gsz):
                    pltpu.sync_copy(accum.at[pl.ds(r0, gsz)],
                                    rowbuf.at[pl.ds(0, gsz)])

                    def fin_row(j, _):
                        for h in range(4):
                            wv = plsc.load_gather(
                                invv, [izero16 + ((rl0 + j) * 16 + h)])
                            for v in range(16):
                                sl2 = pl.ds(h * 256 + v * 16, 16)
                                rowbuf[j, sl2] = rowbuf[j, sl2] * wv
                        return 0

                    lax.fori_loop(0, gsz, fin_row, 0)
                    pltpu.sync_copy(rowbuf.at[pl.ds(0, gsz)],
                                    out_hbm.at[pl.ds(lo + r0, gsz)])

                def emit16(gi, _):
                    emit_grp(s * rt + gi * 16, gi * 16, 16)
                    return 0

                lax.fori_loop(0, rt // 16, emit16, 0)
                if rt % 16:
                    emit_grp(s * rt + (rt // 16) * 16, (rt // 16) * 16,
                             rt % 16)
            else:
                pltpu.sync_copy(accum.at[pl.ds(s * rt, rt)],
                                out_hbm.at[pl.ds(lo + s * rt, rt)])
            plsc.subcore_barrier()
            return 0

        lax.fori_loop(0, _NPASS, pass_body, 0)

    return sc_agg


def kernel(x, edge_index, edge_attr, W1, as1, ad1, We1, ae1, b1,
           W2, as2, ad2, We2, ae2, b2, L1, bL1, L2, bL2):
    n, din = x.shape
    e, edim = edge_attr.shape
    H, C = as1.shape[1], as1.shape[2]
    f32 = jnp.float32

    src = edge_index[0].astype(jnp.int32)
    dst = edge_index[1].astype(jnp.int32)
    src2 = src.reshape(e // _ICH, _ICH)
    dst2 = dst.reshape(e // _ICH, _ICH)

    vs1 = (W1.reshape(din, H, C) * as1).sum(-1)
    vd1 = (W1.reshape(din, H, C) * ad1).sum(-1)
    vsd1 = jnp.concatenate([vs1, vd1], axis=1)
    vsd1 = jnp.pad(vsd1, ((0, 0), (0, 120)))
    hc = H * C
    vs2 = (W2.reshape(hc, H, C) * as2).sum(-1)
    vd2 = (W2.reshape(hc, H, C) * ad2).sum(-1)
    vsd2 = jnp.concatenate([vs2, vd2], axis=1)
    vsd2 = jnp.pad(vsd2, ((0, 0), (0, 120)))
    me1 = (We1.reshape(edim, H, C) * ae1).sum(-1)
    me2 = (We2.reshape(edim, H, C) * ae2).sum(-1)
    mcat = jnp.pad(jnp.concatenate([me1, me2], axis=1),
                   ((0, 0), (0, 8)))
    wblk = jnp.kron(jnp.eye(8, dtype=f32), mcat)

    zrow1024 = jnp.zeros((1, 1024), f32)
    zrow128 = jnp.zeros((1, 128), f32)
    z16 = jnp.zeros((_NPAD, 16), f32)
    zacc = jnp.zeros((_R, 1024), f32)
    zacc256 = jnp.zeros((_R, 256), f32)

    xpad = jnp.concatenate([x, jnp.zeros((_NPAD - n, din), f32)], axis=0)

    eaf = _matmul(edge_attr.reshape(e // 8, 128), wblk, zrow128, bm=2000)
    ea16 = eaf.reshape(e, 16)

    sc_alpha1 = _make_sc_alpha(e, _NPAD, aoff=0)
    sc_alpha2 = _make_sc_alpha(e, _NPAD, aoff=4)
    sc_agg_c = _make_sc_agg(e, _NPAD, concat=True)
    sc_agg_m = _make_sc_agg(e, _NPAD, concat=False)

    xp1 = _matmul(xpad, W1, zrow1024, bm=768)
    ac1 = _matmul(xpad, vsd1, zrow128, bm=768)
    avd1 = ac1[:, 0:16]
    expal1, dnm1 = sc_alpha1(src2, dst2, avd1, ea16, z16)
    h1 = sc_agg_c(src, dst, expal1, dnm1.reshape(_NC, _NPAD * 16), xp1,
                  zacc)

    xp2 = _matmul(h1, W2, (b1 @ W2).reshape(1, hc), bm=768)
    ac2 = _matmul(h1, vsd2, (b1 @ vsd2).reshape(1, 128), bm=768)
    avd2 = ac2[:, 0:16]
    expal2, dnm2 = sc_alpha2(src2, dst2, avd2, ea16, z16)
    h2 = sc_agg_m(src, dst, expal2, dnm2.reshape(_NC, _NPAD * 16), xp2,
                  zacc256)

    out = _head_mlp(h2, L1, (b2 @ L1 + bL1).reshape(1, C),
                    L2, bL2.reshape(1, -1), bm=768)
    return out[:n]
